# baseline (speedup 1.0000x reference)
import jax
import jax.numpy as jnp
from jax import lax
from jax.experimental import pallas as pl
from jax.experimental.pallas import tpu as pltpu
from jax.experimental.pallas import tpu_sc as plsc

_ROWS, _COLS = 16384, 1024
_N = _ROWS * _COLS
_R = _N // 4

_NC, _NS = 2, 16
_NW = _NC * _NS
_CS = 8192
_CSH = 13
_NCHUNK = _N // _CS
_CPW = _NCHUNK // _NW
_SPAN = _CS * _CPW
_SU = 1024
_MAXW = _CS // _SU + 1
_WCAP = _MAXW * _SU
_VPW = _SU // 16
_PSU = 8192
_PVW = _PSU // 16
_BIG = 2 ** 30


def _sc_body(fix_ref, par_ref, idxp_ref, sb_ref, out_ref,
             dbuf, wbi, wbp, pwm, karr, sbv, sem_i, sem_o, sem_p):
    c = lax.axis_index("c")
    s = lax.axis_index("s")
    wid = s * _NC + c
    ch0 = wid * _CPW

    pltpu.sync_copy(sb_ref, sbv)
    lane = lax.iota(jnp.int32, 16)

    kv = sbv[pl.ds(wid, 16)]
    k0t = kv[0]
    k1t = kv[1]
    a0t = k0t & ~jnp.int32(7)

    big16 = jnp.full((16,), _BIG, dtype=jnp.int32)
    for j in range(4):
        karr[pl.ds(j * 16, 16)] = big16
    karr[pl.ds(64, 16)] = jnp.broadcast_to(k1t, (16,)).astype(jnp.int32)

    npre = lax.max(jnp.int32(0), (k1t - a0t + _PSU - 1) // _PSU)

    def pre_start(t, half):
        sft = pl.multiple_of(
            lax.min(a0t + t * _PSU, jnp.int32(_R - _PSU)), 8)
        po = pl.multiple_of(half * (_PSU + 8), 8)
        pltpu.make_async_copy(
            idxp_ref.at[pl.ds(sft, _PSU + 8)],
            pwm.at[pl.ds(po, _PSU + 8)], sem_i).start()

    def pre_wait(half):
        po = pl.multiple_of(half * (_PSU + 8), 8)
        pltpu.make_async_copy(
            idxp_ref.at[pl.ds(0, _PSU + 8)],
            pwm.at[pl.ds(po, _PSU + 8)], sem_i).wait()

    @pl.when(npre > 0)
    def _():
        pre_start(0, 0)

    def pre_body(t, carry):
        half = lax.rem(t, 2)
        base = half * (_PSU + 8)
        sft = pl.multiple_of(
            lax.min(a0t + t * _PSU, jnp.int32(_R - _PSU)), 8)
        pre_wait(half)

        @pl.when(t + 1 < npre)
        def _():
            pre_start(t + 1, 1 - half)

        def pre_vec(v, carry2):
            cur = pwm[pl.ds(base + 8 + v * 16, 16)]
            prv = pwm[pl.ds(base + 7 + v * 16, 16)]
            cid = lax.shift_right_arithmetic(cur, _CSH)
            pid = lax.shift_right_arithmetic(prv, _CSH)
            pos = sft + v * 16 + lane
            m = (cid != pid) & (pos >= k0t) & (pos < k1t)
            plsc.store_scatter(karr, [cid - ch0], pos, mask=m)
            return carry2

        lax.fori_loop(0, _PVW, pre_vec, 0)
        return carry

    lax.fori_loop(0, npre, pre_body, 0)

    carry0 = jnp.int32(_BIG)
    for j in range(4, -1, -1):
        v = karr[pl.ds(j * 16, 16)]
        nm = plsc.cummax(lax.rev(-v, (0,)))
        sv = lax.max(lax.rev(nm, (0,)), -carry0)
        karr[pl.ds(j * 16, 16)] = -sv
        carry0 = -sv[0]

    def seg(cl):
        bv = karr[pl.ds(cl, 16)]
        k0 = bv[0]
        k1 = bv[1]
        a0 = k0 & ~jnp.int32(7)
        nwin = lax.max(jnp.int32(0), (k1 - a0 + _SU - 1) // _SU)
        return k0, k1, a0, nwin

    def win_start(a0, t):
        return pl.multiple_of(
            lax.min(a0 + t * _SU, jnp.int32(_R - _SU)), 8)

    _RPC = _CS // _COLS

    def dest_in_start(cl, b):
        r0 = pl.multiple_of((wid * _SPAN + cl * _CS) // _COLS, 8)
        br = pl.multiple_of(b * _RPC, 8)
        pltpu.make_async_copy(
            fix_ref.at[pl.ds(r0, _RPC), :], dbuf.at[pl.ds(br, _RPC), :],
            sem_i).start()

    def dest_out_start(cl, b):
        r0 = pl.multiple_of((wid * _SPAN + cl * _CS) // _COLS, 8)
        br = pl.multiple_of(b * _RPC, 8)
        pltpu.make_async_copy(
            dbuf.at[pl.ds(br, _RPC), :], out_ref.at[pl.ds(r0, _RPC), :],
            sem_o).start()

    def dest_in_wait(b):
        br = pl.multiple_of(b * _RPC, 8)
        pltpu.make_async_copy(
            fix_ref.at[pl.ds(0, _RPC), :], dbuf.at[pl.ds(br, _RPC), :],
            sem_i).wait()

    def dest_out_wait(b):
        br = pl.multiple_of(b * _RPC, 8)
        pltpu.make_async_copy(
            dbuf.at[pl.ds(br, _RPC), :], out_ref.at[pl.ds(0, _RPC), :],
            sem_o).wait()

    def param_fetch(cl, slot):
        _, _, a0, nwin = seg(cl)
        for t in range(_MAXW):
            @pl.when(t < nwin)
            def _():
                sft = win_start(a0, t)
                wo = pl.multiple_of(slot * _WCAP + t * _SU, 8)
                pltpu.make_async_copy(
                    idxp_ref.at[pl.ds(sft + 8, _SU)],
                    wbi.at[pl.ds(wo, _SU)], sem_p).start()
                pltpu.make_async_copy(
                    par_ref.at[pl.ds(sft, _SU)],
                    wbp.at[pl.ds(wo, _SU)], sem_p).start()

    def param_drain(nwin, slot):
        for t in range(_MAXW):
            @pl.when(t < nwin)
            def _():
                wo = pl.multiple_of(slot * _WCAP + t * _SU, 8)
                pltpu.make_async_copy(
                    idxp_ref.at[pl.ds(8, _SU)],
                    wbi.at[pl.ds(wo, _SU)], sem_p).wait()
                pltpu.make_async_copy(
                    par_ref.at[pl.ds(0, _SU)],
                    wbp.at[pl.ds(wo, _SU)], sem_p).wait()

    dest_in_start(0, 0)
    dest_in_start(1, 1)
    param_fetch(0, 0)

    def chunk_body(cl, carry):
        b = lax.rem(cl, 4)
        slot = lax.rem(cl, 2)
        k0, k1, a0, nwin = seg(cl)
        off = b * _CS - (wid * _SPAN + cl * _CS)

        dest_in_wait(b)
        param_drain(nwin, slot)

        @pl.when(cl >= 2)
        def _():
            dest_out_wait(lax.rem(cl + 2, 4))

        @pl.when(cl + 2 < _CPW)
        def _():
            dest_in_start(cl + 2, lax.rem(cl + 2, 4))

        @pl.when(cl + 1 < _CPW)
        def _():
            param_fetch(cl + 1, 1 - slot)

        for t in range(_MAXW):
            sft = win_start(a0, t)
            vhi = lax.select(
                t < nwin,
                lax.clamp(jnp.int32(0), (k1 - sft + 15) // 16,
                          jnp.int32(_VPW)),
                jnp.int32(0))

            def vec_body(v, carry3, t=t, sft=sft):
                pos = sft + v * 16 + lane
                ivec = wbi[pl.ds(slot * _WCAP + t * _SU + v * 16, 16)]
                pvec = wbp[pl.ds(slot * _WCAP + t * _SU + v * 16, 16)]
                mask = (pos >= k0) & (pos < k1)
                rel = ivec + off
                plsc.store_scatter(
                    dbuf,
                    [lax.shift_right_logical(rel, 10),
                     lax.bitwise_and(rel, jnp.int32(_COLS - 1))],
                    pvec, mask=mask)
                return carry3

            lax.fori_loop(0, vhi, vec_body, 0)

        dest_out_start(cl, b)
        return carry

    lax.fori_loop(0, _CPW, chunk_body, 0)

    dest_out_wait((_CPW - 2) % 4)
    dest_out_wait((_CPW - 1) % 4)


def _make_sc_kernel():
    mesh = plsc.VectorSubcoreMesh(
        core_axis_name="c", subcore_axis_name="s",
        num_cores=_NC, num_subcores=_NS,
    )
    return pl.kernel(
        _sc_body,
        out_type=jax.ShapeDtypeStruct((_ROWS, _COLS), jnp.float32),
        mesh=mesh,
        scratch_types=[
            pltpu.VMEM((4 * _CS // _COLS, _COLS), jnp.float32),
            pltpu.VMEM((2 * _WCAP,), jnp.int32),
            pltpu.VMEM((2 * _WCAP,), jnp.float32),
            pltpu.VMEM((2 * (_PSU + 8),), jnp.int32),
            pltpu.VMEM((80,), jnp.int32),
            pltpu.VMEM((48,), jnp.int32),
            pltpu.SemaphoreType.DMA,
            pltpu.SemaphoreType.DMA,
            pltpu.SemaphoreType.DMA,
        ],
        compiler_params=pltpu.CompilerParams(needs_layout_passes=False),
    )


def kernel(fixed_values, refinable_params, refinable_idx):
    idx32 = refinable_idx.astype(jnp.int32)
    idxp = jnp.concatenate(
        [jnp.full((8,), -1, dtype=jnp.int32), idx32])
    cuts = jnp.arange(_NW + 1, dtype=jnp.int32) * _SPAN
    sb = jnp.searchsorted(idx32, cuts, side="left").astype(jnp.int32)
    sb = jnp.concatenate(
        [sb, jnp.full((48 - _NW - 1,), _R, dtype=jnp.int32)])
    return _make_sc_kernel()(fixed_values, refinable_params, idxp, sb)


# speedup vs baseline: 1.0170x; 1.0170x over previous
import jax
import jax.numpy as jnp
from jax import lax
from jax.experimental import pallas as pl
from jax.experimental.pallas import tpu as pltpu
from jax.experimental.pallas import tpu_sc as plsc

_ROWS, _COLS = 16384, 1024
_N = _ROWS * _COLS
_R = _N // 4

_NC, _NS = 2, 16
_NW = _NC * _NS
_CS = 8192
_CSH = 13
_NCHUNK = _N // _CS
_CPW = _NCHUNK // _NW
_SPAN = _CS * _CPW
_SU = 2048
_MAXW = _CS // _SU + 1
_WCAP = _MAXW * _SU
_VPW = _SU // 16
_PSU = 8192
_PVW = _PSU // 16
_BIG = 2 ** 30


def _sc_body(fix_ref, par_ref, idxp_ref, sb_ref, out_ref,
             dbuf, wbi, wbp, pwm, karr, sbv, sem_i, sem_o, sem_p):
    c = lax.axis_index("c")
    s = lax.axis_index("s")
    wid = s * _NC + c
    ch0 = wid * _CPW

    pltpu.sync_copy(sb_ref, sbv)
    lane = lax.iota(jnp.int32, 16)

    kv = sbv[pl.ds(wid, 16)]
    k0t = kv[0]
    k1t = kv[1]
    a0t = k0t & ~jnp.int32(7)

    big16 = jnp.full((16,), _BIG, dtype=jnp.int32)
    for j in range(4):
        karr[pl.ds(j * 16, 16)] = big16
    karr[pl.ds(64, 16)] = jnp.broadcast_to(k1t, (16,)).astype(jnp.int32)

    npre = lax.max(jnp.int32(0), (k1t - a0t + _PSU - 1) // _PSU)

    def pre_start(t, half):
        sft = pl.multiple_of(
            lax.min(a0t + t * _PSU, jnp.int32(_R - _PSU)), 8)
        po = pl.multiple_of(half * (_PSU + 8), 8)
        pltpu.make_async_copy(
            idxp_ref.at[pl.ds(sft, _PSU + 8)],
            pwm.at[pl.ds(po, _PSU + 8)], sem_i).start()

    def pre_wait(half):
        po = pl.multiple_of(half * (_PSU + 8), 8)
        pltpu.make_async_copy(
            idxp_ref.at[pl.ds(0, _PSU + 8)],
            pwm.at[pl.ds(po, _PSU + 8)], sem_i).wait()

    @pl.when(npre > 0)
    def _():
        pre_start(0, 0)

    def pre_body(t, carry):
        half = lax.rem(t, 2)
        base = half * (_PSU + 8)
        sft = pl.multiple_of(
            lax.min(a0t + t * _PSU, jnp.int32(_R - _PSU)), 8)
        pre_wait(half)

        @pl.when(t + 1 < npre)
        def _():
            pre_start(t + 1, 1 - half)

        def pre_vec(v, carry2):
            cur = pwm[pl.ds(base + 8 + v * 16, 16)]
            prv = pwm[pl.ds(base + 7 + v * 16, 16)]
            cid = lax.shift_right_arithmetic(cur, _CSH)
            pid = lax.shift_right_arithmetic(prv, _CSH)
            pos = sft + v * 16 + lane
            m = (cid != pid) & (pos >= k0t) & (pos < k1t)
            plsc.store_scatter(karr, [cid - ch0], pos, mask=m)
            return carry2

        lax.fori_loop(0, _PVW, pre_vec, 0)
        return carry

    lax.fori_loop(0, npre, pre_body, 0)

    carry0 = jnp.int32(_BIG)
    for j in range(4, -1, -1):
        v = karr[pl.ds(j * 16, 16)]
        nm = plsc.cummax(lax.rev(-v, (0,)))
        sv = lax.max(lax.rev(nm, (0,)), -carry0)
        karr[pl.ds(j * 16, 16)] = -sv
        carry0 = -sv[0]

    def seg(cl):
        bv = karr[pl.ds(cl, 16)]
        k0 = bv[0]
        k1 = bv[1]
        a0 = k0 & ~jnp.int32(7)
        nwin = lax.max(jnp.int32(0), (k1 - a0 + _SU - 1) // _SU)
        return k0, k1, a0, nwin

    def win_start(a0, t):
        return pl.multiple_of(
            lax.min(a0 + t * _SU, jnp.int32(_R - _SU)), 8)

    _RPC = _CS // _COLS

    def dest_in_start(cl, b):
        r0 = pl.multiple_of((wid * _SPAN + cl * _CS) // _COLS, 8)
        br = pl.multiple_of(b * _RPC, 8)
        pltpu.make_async_copy(
            fix_ref.at[pl.ds(r0, _RPC), :], dbuf.at[pl.ds(br, _RPC), :],
            sem_i).start()

    def dest_out_start(cl, b):
        r0 = pl.multiple_of((wid * _SPAN + cl * _CS) // _COLS, 8)
        br = pl.multiple_of(b * _RPC, 8)
        pltpu.make_async_copy(
            dbuf.at[pl.ds(br, _RPC), :], out_ref.at[pl.ds(r0, _RPC), :],
            sem_o).start()

    def dest_in_wait(b):
        br = pl.multiple_of(b * _RPC, 8)
        pltpu.make_async_copy(
            fix_ref.at[pl.ds(0, _RPC), :], dbuf.at[pl.ds(br, _RPC), :],
            sem_i).wait()

    def dest_out_wait(b):
        br = pl.multiple_of(b * _RPC, 8)
        pltpu.make_async_copy(
            dbuf.at[pl.ds(br, _RPC), :], out_ref.at[pl.ds(0, _RPC), :],
            sem_o).wait()

    def param_fetch(cl, slot):
        _, _, a0, nwin = seg(cl)
        for t in range(_MAXW):
            @pl.when(t < nwin)
            def _():
                sft = win_start(a0, t)
                wo = pl.multiple_of(slot * _WCAP + t * _SU, 8)
                pltpu.make_async_copy(
                    idxp_ref.at[pl.ds(sft + 8, _SU)],
                    wbi.at[pl.ds(wo, _SU)], sem_p).start()
                pltpu.make_async_copy(
                    par_ref.at[pl.ds(sft, _SU)],
                    wbp.at[pl.ds(wo, _SU)], sem_p).start()

    def param_drain(nwin, slot):
        for t in range(_MAXW):
            @pl.when(t < nwin)
            def _():
                wo = pl.multiple_of(slot * _WCAP + t * _SU, 8)
                pltpu.make_async_copy(
                    idxp_ref.at[pl.ds(8, _SU)],
                    wbi.at[pl.ds(wo, _SU)], sem_p).wait()
                pltpu.make_async_copy(
                    par_ref.at[pl.ds(0, _SU)],
                    wbp.at[pl.ds(wo, _SU)], sem_p).wait()

    dest_in_start(0, 0)
    dest_in_start(1, 1)
    param_fetch(0, 0)

    def chunk_body(cl, carry):
        b = lax.rem(cl, 4)
        slot = lax.rem(cl, 2)
        k0, k1, a0, nwin = seg(cl)
        off = b * _CS - (wid * _SPAN + cl * _CS)

        dest_in_wait(b)
        param_drain(nwin, slot)

        @pl.when(cl >= 2)
        def _():
            dest_out_wait(lax.rem(cl + 2, 4))

        @pl.when(cl + 2 < _CPW)
        def _():
            dest_in_start(cl + 2, lax.rem(cl + 2, 4))

        @pl.when(cl + 1 < _CPW)
        def _():
            param_fetch(cl + 1, 1 - slot)

        for t in range(_MAXW):
            sft = win_start(a0, t)
            vhi = lax.select(
                t < nwin,
                lax.clamp(jnp.int32(0), (k1 - sft + 15) // 16,
                          jnp.int32(_VPW)),
                jnp.int32(0))

            def vec_body(v, carry3, t=t, sft=sft):
                pos = sft + v * 16 + lane
                ivec = wbi[pl.ds(slot * _WCAP + t * _SU + v * 16, 16)]
                pvec = wbp[pl.ds(slot * _WCAP + t * _SU + v * 16, 16)]
                mask = (pos >= k0) & (pos < k1)
                rel = ivec + off
                plsc.store_scatter(
                    dbuf,
                    [lax.shift_right_logical(rel, 10),
                     lax.bitwise_and(rel, jnp.int32(_COLS - 1))],
                    pvec, mask=mask)
                return carry3

            lax.fori_loop(0, vhi, vec_body, 0)

        dest_out_start(cl, b)
        return carry

    lax.fori_loop(0, _CPW, chunk_body, 0)

    dest_out_wait((_CPW - 2) % 4)
    dest_out_wait((_CPW - 1) % 4)


def _make_sc_kernel():
    mesh = plsc.VectorSubcoreMesh(
        core_axis_name="c", subcore_axis_name="s",
        num_cores=_NC, num_subcores=_NS,
    )
    return pl.kernel(
        _sc_body,
        out_type=jax.ShapeDtypeStruct((_ROWS, _COLS), jnp.float32),
        mesh=mesh,
        scratch_types=[
            pltpu.VMEM((4 * _CS // _COLS, _COLS), jnp.float32),
            pltpu.VMEM((2 * _WCAP,), jnp.int32),
            pltpu.VMEM((2 * _WCAP,), jnp.float32),
            pltpu.VMEM((2 * (_PSU + 8),), jnp.int32),
            pltpu.VMEM((80,), jnp.int32),
            pltpu.VMEM((48,), jnp.int32),
            pltpu.SemaphoreType.DMA,
            pltpu.SemaphoreType.DMA,
            pltpu.SemaphoreType.DMA,
        ],
        compiler_params=pltpu.CompilerParams(needs_layout_passes=False),
    )


def kernel(fixed_values, refinable_params, refinable_idx):
    idx32 = refinable_idx.astype(jnp.int32)
    idxp = jnp.concatenate(
        [jnp.full((8,), -1, dtype=jnp.int32), idx32])
    cuts = jnp.arange(_NW + 1, dtype=jnp.int32) * _SPAN
    sb = jnp.searchsorted(idx32, cuts, side="left").astype(jnp.int32)
    sb = jnp.concatenate(
        [sb, jnp.full((48 - _NW - 1,), _R, dtype=jnp.int32)])
    return _make_sc_kernel()(fixed_values, refinable_params, idxp, sb)
